# baseline (speedup 1.0000x reference)
import functools

import jax
import jax.numpy as jnp
from jax import lax
from jax.experimental import pallas as pl
from jax.experimental.pallas import tpu as pltpu
from jax.experimental.pallas import tpu_sc as plsc


def _v_matmul(internal_flat, w):
    n, d = internal_flat.shape

    def body(x_ref, w_ref, o_ref):
        o_ref[...] = jnp.dot(x_ref[...], w_ref[...],
                             preferred_element_type=jnp.float32)

    return pl.pallas_call(
        body,
        out_shape=jax.ShapeDtypeStruct((n, d), jnp.float32),
    )(internal_flat, w)


def _sc_aggregate(ts, table, v_flat, t_max, d):
    ntok = ts.shape[0]
    tblk = table.shape[1]
    nextn = tblk // d
    ndg = d // 16
    nng = nextn // 16
    nw = 32
    tpw = ntok // nw

    mesh = plsc.VectorSubcoreMesh(core_axis_name="c", subcore_axis_name="s")

    @functools.partial(
        pl.kernel, mesh=mesh,
        compiler_params=pltpu.CompilerParams(needs_layout_passes=False),
        out_type=jax.ShapeDtypeStruct((ntok * d,), jnp.float32),
        scratch_types=[
            pltpu.VMEM((ntok + 32,), jnp.int32),
            pltpu.VMEM((32 * 8,), jnp.int32),
            pltpu.VMEM((1, tblk), jnp.float32),
            pltpu.VMEM((1, tblk), jnp.float32),
            pltpu.VMEM((tpw * d,), jnp.float32),
            pltpu.VMEM((nextn,), jnp.float32),
            pltpu.VMEM((tpw * d,), jnp.float32),
            pltpu.SemaphoreType.DMA,
            pltpu.SemaphoreType.DMA,
        ],
    )
    def body(ts_hbm, table_hbm, v_hbm, out_hbm,
             ts_v, ids8, ebuf_a, ebuf_b, vrows, att_v, obuf, sem_a, sem_b):
        nc = 2
        wid = lax.axis_index("s") * nc + lax.axis_index("c")
        base = wid * tpw
        iota = lax.iota(jnp.int32, 16)
        zero16i = jnp.zeros((16,), jnp.int32)
        zero16f = jnp.zeros((16,), jnp.float32)

        pltpu.sync_copy(ts_hbm, ts_v.at[pl.ds(0, ntok)])
        for it in range(2):
            tok = it * 16 + iota
            raw = plsc.load_gather(ts_v, [base + tok])
            plsc.store_scatter(ids8, [tok * 8], jnp.clip(raw, 0, t_max - 1))

        pltpu.sync_copy(v_hbm.at[pl.ds(base * d, tpw * d)], vrows)

        bufs = (ebuf_a, ebuf_b)
        sems = (sem_a, sem_b)
        copies = [None, None]

        def start(i):
            b = i % 2
            copies[b] = pltpu.async_copy(
                table_hbm.at[ids8.at[pl.ds(i * 8, 1)]], bufs[b], sems[b])

        start(0)

        for i in range(tpw):
            b = i % 2
            if i + 1 < tpw:
                start(i + 1)
            copies[b].wait()
            ebuf = bufs[b]

            def dot_body(dd, carry):
                accs, idxs, dsplat = carry
                vd = plsc.load_gather(vrows, [dsplat])
                new_accs = tuple(
                    accs[ng] + plsc.load_gather(ebuf, [zero16i, idxs[ng]]) * vd
                    for ng in range(nng))
                new_idxs = tuple(idxs[ng] + 1 for ng in range(nng))
                return (new_accs, new_idxs, dsplat + 1)

            init_idxs = tuple((iota + ng * 16) * d for ng in range(nng))
            accs, _, _ = lax.fori_loop(
                0, d, dot_body,
                (tuple([zero16f] * nng), init_idxs,
                 jnp.full((16,), i * d, jnp.int32)))

            m8 = accs[0]
            for ng in range(1, nng):
                m8 = jnp.maximum(m8, accs[ng])
            m = jnp.max(m8)
            exps = [jnp.exp(a - m) for a in accs]
            s8 = exps[0]
            for ng in range(1, nng):
                s8 = s8 + exps[ng]
            s = jnp.sum(s8)
            for ng in range(nng):
                att_v[pl.ds(ng * 16, 16)] = exps[ng] / s

            def agg_body(n, carry):
                aggs, nsplat, off = carry
                an = plsc.load_gather(att_v, [nsplat])
                new_aggs = tuple(
                    aggs[dg] + an * ebuf[0, pl.ds(off + dg * 16, 16)]
                    for dg in range(ndg))
                return (new_aggs, nsplat + 1, off + d)

            aggs, _, _ = lax.fori_loop(
                0, nextn, agg_body, (tuple([zero16f] * ndg), zero16i, 0))
            for dg in range(ndg):
                obuf[pl.ds(i * d + dg * 16, 16)] = aggs[dg]

        pltpu.sync_copy(obuf, out_hbm.at[pl.ds(base * d, tpw * d)])

    return body(ts, table, v_flat)


def kernel(internal_emb, timestamps, time_list, ext_embeddings,
           time_to_embeddings, W_att, b_att):
    bs, seq, d = internal_emb.shape
    t_max, nextn, _ = ext_embeddings.shape
    internal_flat = internal_emb.reshape(bs * seq, d)
    ts_flat = timestamps.reshape(-1).astype(jnp.int32)
    table = ext_embeddings.reshape(t_max, nextn * d)
    v = _v_matmul(internal_flat, W_att)
    out = _sc_aggregate(ts_flat, table, v.reshape(-1), t_max, d)
    return out.reshape(bs, seq, d)


# speedup vs baseline: 2.1809x; 2.1809x over previous
import functools

import jax
import jax.numpy as jnp
from jax import lax
from jax.experimental import pallas as pl
from jax.experimental.pallas import tpu as pltpu
from jax.experimental.pallas import tpu_sc as plsc


def _v_matmul(internal_flat, w):
    n, d = internal_flat.shape

    def body(x_ref, w_ref, o_ref):
        o_ref[...] = jnp.dot(x_ref[...], w_ref[...],
                             preferred_element_type=jnp.float32)

    return pl.pallas_call(
        body,
        out_shape=jax.ShapeDtypeStruct((n, d), jnp.float32),
    )(internal_flat, w)


def _sc_aggregate(ts, table, v_flat, t_max, d):
    ntok = ts.shape[0]
    tblk = table.shape[1]
    nextn = tblk // d
    ndg = d // 16
    nng = nextn // 16
    nw = 32
    tpw = ntok // nw

    mesh = plsc.VectorSubcoreMesh(core_axis_name="c", subcore_axis_name="s")

    @functools.partial(
        pl.kernel, mesh=mesh,
        compiler_params=pltpu.CompilerParams(needs_layout_passes=False),
        out_type=jax.ShapeDtypeStruct((ntok * d,), jnp.float32),
        scratch_types=[
            pltpu.VMEM((ntok + 32,), jnp.int32),
            pltpu.VMEM((32 * 8,), jnp.int32),
            pltpu.VMEM((1, tblk), jnp.float32),
            pltpu.VMEM((1, tblk), jnp.float32),
            pltpu.VMEM((tpw * d,), jnp.float32),
            pltpu.VMEM((nextn,), jnp.float32),
            pltpu.VMEM((16 * 17,), jnp.float32),
            pltpu.VMEM((tpw * d,), jnp.float32),
            pltpu.SemaphoreType.DMA,
            pltpu.SemaphoreType.DMA,
        ],
    )
    def body(ts_hbm, table_hbm, v_hbm, out_hbm,
             ts_v, ids8, ebuf_a, ebuf_b, vrows, att_v, pbuf, obuf,
             sem_a, sem_b):
        nc = 2
        wid = lax.axis_index("s") * nc + lax.axis_index("c")
        base = wid * tpw
        iota = lax.iota(jnp.int32, 16)
        zero16i = jnp.zeros((16,), jnp.int32)
        zero16f = jnp.zeros((16,), jnp.float32)

        pltpu.sync_copy(ts_hbm, ts_v.at[pl.ds(0, ntok)])
        for it in range(2):
            tok = it * 16 + iota
            raw = plsc.load_gather(ts_v, [base + tok])
            plsc.store_scatter(ids8, [tok * 8], jnp.clip(raw, 0, t_max - 1))

        pltpu.sync_copy(v_hbm.at[pl.ds(base * d, tpw * d)], vrows)

        bufs = (ebuf_a, ebuf_b)
        sems = (sem_a, sem_b)

        def start(t, b):
            pltpu.async_copy(
                table_hbm.at[ids8.at[pl.ds(pl.multiple_of(t * 8, 8), 1)]],
                bufs[b], sems[b])

        def wait(b):
            pltpu.make_async_copy(
                table_hbm.at[pl.ds(0, 1)], bufs[b], sems[b]).wait()

        skew_idx = iota * 17

        def compute_token(t, ebuf):
            vvecs = [vrows[pl.ds(t * d + dg * 16, 16)] for dg in range(ndg)]

            def dot_body(nb, carry):
                off = nb * 16 * d
                for j in range(16):
                    pacc = ebuf[0, pl.ds(off + j * d, 16)] * vvecs[0]
                    for dg in range(1, ndg):
                        pacc = pacc + (
                            ebuf[0, pl.ds(off + j * d + dg * 16, 16)]
                            * vvecs[dg])
                    pbuf[pl.ds(j * 17, 16)] = pacc
                s = plsc.load_gather(pbuf, [skew_idx])
                for l in range(1, 16):
                    s = s + plsc.load_gather(pbuf, [skew_idx + l])
                att_v[pl.ds(nb * 16, 16)] = s
                return carry

            lax.fori_loop(0, nng, dot_body, 0)

            accs = [att_v[pl.ds(ng * 16, 16)] for ng in range(nng)]
            m8 = accs[0]
            for ng in range(1, nng):
                m8 = jnp.maximum(m8, accs[ng])
            m = jnp.max(m8)
            exps = [jnp.exp(a - m) for a in accs]
            s8 = exps[0]
            for ng in range(1, nng):
                s8 = s8 + exps[ng]
            s = jnp.sum(s8)
            for ng in range(nng):
                att_v[pl.ds(ng * 16, 16)] = exps[ng] / s

            def agg_body(nb, aggs):
                ablk = att_v[pl.ds(nb * 16, 16)]
                off = nb * 16 * d
                for j in range(16):
                    an = ablk[j]
                    aggs = tuple(
                        aggs[dg] + an * ebuf[0, pl.ds(off + j * d + dg * 16, 16)]
                        for dg in range(ndg))
                return aggs

            aggs = lax.fori_loop(
                0, nng, agg_body, tuple([zero16f] * ndg))
            for dg in range(ndg):
                obuf[pl.ds(t * d + dg * 16, 16)] = aggs[dg]

        start(0, 0)
        start(1, 1)

        def pair_body(g, carry):
            t0 = 2 * g
            wait(0)
            compute_token(t0, ebuf_a)
            start(t0 + 2, 0)
            wait(1)
            compute_token(t0 + 1, ebuf_b)

            @pl.when(t0 + 3 < tpw)
            def _():
                start(t0 + 3, 1)

            return carry

        lax.fori_loop(0, (tpw - 1) // 2, pair_body, 0)
        wait(0)
        compute_token(tpw - 1, ebuf_a)

        pltpu.sync_copy(obuf, out_hbm.at[pl.ds(base * d, tpw * d)])

    return body(ts, table, v_flat)


def kernel(internal_emb, timestamps, time_list, ext_embeddings,
           time_to_embeddings, W_att, b_att):
    bs, seq, d = internal_emb.shape
    t_max, nextn, _ = ext_embeddings.shape
    internal_flat = internal_emb.reshape(bs * seq, d)
    ts_flat = timestamps.reshape(-1).astype(jnp.int32)
    table = ext_embeddings.reshape(t_max, nextn * d)
    v = _v_matmul(internal_flat, W_att)
    out = _sc_aggregate(ts_flat, table, v.reshape(-1), t_max, d)
    return out.reshape(bs, seq, d)


# speedup vs baseline: 3.2903x; 1.5087x over previous
import functools

import jax
import jax.numpy as jnp
from jax import lax
from jax.experimental import pallas as pl
from jax.experimental.pallas import tpu as pltpu
from jax.experimental.pallas import tpu_sc as plsc


def _v_matmul(internal_flat, w):
    n, d = internal_flat.shape

    def body(x_ref, w_ref, o_ref):
        o_ref[...] = jnp.dot(x_ref[...], w_ref[...],
                             preferred_element_type=jnp.float32)

    return pl.pallas_call(
        body,
        out_shape=jax.ShapeDtypeStruct((n, d), jnp.float32),
    )(internal_flat, w)


def _sc_aggregate(ts, table, v_flat, t_max, d):
    ntok = ts.shape[0]
    nextn = table.shape[1]
    ndg = d // 16
    nng = nextn // 16
    nw = 32
    tpw = ntok // nw

    mesh = plsc.VectorSubcoreMesh(core_axis_name="c", subcore_axis_name="s")

    @functools.partial(
        pl.kernel, mesh=mesh,
        compiler_params=pltpu.CompilerParams(needs_layout_passes=False),
        out_type=jax.ShapeDtypeStruct((ntok * d,), jnp.float32),
        scratch_types=[
            pltpu.VMEM((ntok + 32,), jnp.int32),
            pltpu.VMEM((32 * 8,), jnp.int32),
            pltpu.VMEM((1, nextn, d), jnp.float32),
            pltpu.VMEM((1, nextn, d), jnp.float32),
            pltpu.VMEM((tpw * d,), jnp.float32),
            pltpu.VMEM((nextn,), jnp.float32),
            pltpu.VMEM((16 * 17,), jnp.float32),
            pltpu.VMEM((tpw * d,), jnp.float32),
            pltpu.SemaphoreType.DMA,
            pltpu.SemaphoreType.DMA,
        ],
    )
    def body(ts_hbm, table_hbm, v_hbm, out_hbm,
             ts_v, ids8, ebuf_a, ebuf_b, vrows, att_v, pbuf, obuf,
             sem_a, sem_b):
        nc = 2
        wid = lax.axis_index("s") * nc + lax.axis_index("c")
        base = wid * tpw
        iota = lax.iota(jnp.int32, 16)
        zero16i = jnp.zeros((16,), jnp.int32)
        zero16f = jnp.zeros((16,), jnp.float32)

        pltpu.sync_copy(ts_hbm, ts_v.at[pl.ds(0, ntok)])
        for it in range(2):
            tok = it * 16 + iota
            raw = plsc.load_gather(ts_v, [base + tok])
            plsc.store_scatter(ids8, [tok * 8], jnp.clip(raw, 0, t_max - 1))

        pltpu.sync_copy(v_hbm.at[pl.ds(base * d, tpw * d)], vrows)

        bufs = (ebuf_a, ebuf_b)
        sems = (sem_a, sem_b)

        def start(t, b):
            pltpu.async_copy(
                table_hbm.at[ids8.at[pl.ds(pl.multiple_of(t * 8, 8), 1)]],
                bufs[b], sems[b])

        def wait(b):
            pltpu.make_async_copy(
                table_hbm.at[pl.ds(0, 1)], bufs[b], sems[b]).wait()

        skew_idx = iota * 17

        def compute_token(t, ebuf):
            vvecs = [vrows[pl.ds(t * d + dg * 16, 16)] for dg in range(ndg)]

            def dot_body(nb, carry):
                n0 = nb * 16
                for j in range(16):
                    pacc = ebuf[0, n0 + j, pl.ds(0, 16)] * vvecs[0]
                    for dg in range(1, ndg):
                        pacc = pacc + (
                            ebuf[0, n0 + j, pl.ds(dg * 16, 16)]
                            * vvecs[dg])
                    pbuf[pl.ds(j * 17, 16)] = pacc
                s = plsc.load_gather(pbuf, [skew_idx])
                for l in range(1, 16):
                    s = s + plsc.load_gather(pbuf, [skew_idx + l])
                att_v[pl.ds(nb * 16, 16)] = s
                return carry

            lax.fori_loop(0, nng, dot_body, 0)

            accs = [att_v[pl.ds(ng * 16, 16)] for ng in range(nng)]
            m8 = accs[0]
            for ng in range(1, nng):
                m8 = jnp.maximum(m8, accs[ng])
            m = jnp.max(m8)
            exps = [jnp.exp(a - m) for a in accs]
            s8 = exps[0]
            for ng in range(1, nng):
                s8 = s8 + exps[ng]
            s = jnp.sum(s8)
            for ng in range(nng):
                att_v[pl.ds(ng * 16, 16)] = exps[ng] / s

            def agg_body(nb, aggs):
                ablk = att_v[pl.ds(nb * 16, 16)]
                n0 = nb * 16
                for j in range(16):
                    an = ablk[j]
                    aggs = tuple(
                        aggs[dg] + an * ebuf[0, n0 + j, pl.ds(dg * 16, 16)]
                        for dg in range(ndg))
                return aggs

            aggs = lax.fori_loop(
                0, nng, agg_body, tuple([zero16f] * ndg))
            for dg in range(ndg):
                obuf[pl.ds(t * d + dg * 16, 16)] = aggs[dg]

        start(0, 0)
        start(1, 1)

        def pair_body(g, carry):
            t0 = 2 * g
            wait(0)
            compute_token(t0, ebuf_a)
            start(t0 + 2, 0)
            wait(1)
            compute_token(t0 + 1, ebuf_b)

            @pl.when(t0 + 3 < tpw)
            def _():
                start(t0 + 3, 1)

            return carry

        lax.fori_loop(0, (tpw - 1) // 2, pair_body, 0)
        wait(0)
        compute_token(tpw - 1, ebuf_a)

        pltpu.sync_copy(obuf, out_hbm.at[pl.ds(base * d, tpw * d)])

    return body(ts, table, v_flat)


def kernel(internal_emb, timestamps, time_list, ext_embeddings,
           time_to_embeddings, W_att, b_att):
    bs, seq, d = internal_emb.shape
    t_max, nextn, _ = ext_embeddings.shape
    internal_flat = internal_emb.reshape(bs * seq, d)
    ts_flat = timestamps.reshape(-1).astype(jnp.int32)
    v = _v_matmul(internal_flat, W_att)
    out = _sc_aggregate(ts_flat, ext_embeddings, v.reshape(-1), t_max, d)
    return out.reshape(bs, seq, d)


# speedup vs baseline: 3.4601x; 1.0516x over previous
import functools

import jax
import jax.numpy as jnp
from jax import lax
from jax.experimental import pallas as pl
from jax.experimental.pallas import tpu as pltpu
from jax.experimental.pallas import tpu_sc as plsc


def _v_matmul(internal_flat, w):
    n, d = internal_flat.shape

    def body(x_ref, w_ref, o_ref):
        o_ref[...] = jnp.dot(x_ref[...], w_ref[...],
                             preferred_element_type=jnp.float32)

    return pl.pallas_call(
        body,
        out_shape=jax.ShapeDtypeStruct((n, d), jnp.float32),
    )(internal_flat, w)


def _sc_aggregate(ts, table, v_flat, t_max, d):
    ntok = ts.shape[0]
    nextn = table.shape[1]
    ndg = d // 16
    nng = nextn // 16
    nw = 32
    tpw = ntok // nw

    mesh = plsc.VectorSubcoreMesh(core_axis_name="c", subcore_axis_name="s")

    @functools.partial(
        pl.kernel, mesh=mesh,
        compiler_params=pltpu.CompilerParams(needs_layout_passes=False),
        out_type=jax.ShapeDtypeStruct((ntok * d,), jnp.float32),
        scratch_types=[
            pltpu.VMEM((ntok + 32,), jnp.int32),
            pltpu.VMEM((32 * 8,), jnp.int32),
            pltpu.VMEM((1, nextn, d), jnp.float32),
            pltpu.VMEM((1, nextn, d), jnp.float32),
            pltpu.VMEM((tpw * d,), jnp.float32),
            pltpu.VMEM((nextn,), jnp.float32),
            pltpu.VMEM((16 * 17,), jnp.float32),
            pltpu.VMEM((tpw * d,), jnp.float32),
            pltpu.SemaphoreType.DMA,
            pltpu.SemaphoreType.DMA,
        ],
    )
    def body(ts_hbm, table_hbm, v_hbm, out_hbm,
             ts_v, ids8, ebuf_a, ebuf_b, vrows, att_v, pbuf, obuf,
             sem_a, sem_b):
        nc = 2
        wid = lax.axis_index("s") * nc + lax.axis_index("c")
        base = wid * tpw
        iota = lax.iota(jnp.int32, 16)
        zero16i = jnp.zeros((16,), jnp.int32)
        zero16f = jnp.zeros((16,), jnp.float32)

        pltpu.sync_copy(ts_hbm, ts_v.at[pl.ds(0, ntok)])
        for it in range(2):
            tok = it * 16 + iota
            raw = plsc.load_gather(ts_v, [base + tok])
            plsc.store_scatter(ids8, [tok * 8], jnp.clip(raw, 0, t_max - 1))

        pltpu.sync_copy(v_hbm.at[pl.ds(base * d, tpw * d)], vrows)

        bufs = (ebuf_a, ebuf_b)
        sems = (sem_a, sem_b)

        def start(t, b):
            pltpu.async_copy(
                table_hbm.at[ids8.at[pl.ds(pl.multiple_of(t * 8, 8), 1)]],
                bufs[b], sems[b])

        def wait(b):
            pltpu.make_async_copy(
                table_hbm.at[pl.ds(0, 1)], bufs[b], sems[b]).wait()

        skew_idx = iota * 17

        def compute_token(t, ebuf):
            vvecs = [vrows[pl.ds(t * d + dg * 16, 16)] for dg in range(ndg)]

            def dot_body(nb, carry):
                n0 = nb * 16
                for j in range(16):
                    prods = [
                        ebuf[0, n0 + j, pl.ds(dg * 16, 16)] * vvecs[dg]
                        for dg in range(ndg)]
                    while len(prods) > 1:
                        prods = [a + b for a, b in zip(prods[::2], prods[1::2])]
                    pbuf[pl.ds(j * 17, 16)] = prods[0]
                cols = [plsc.load_gather(pbuf, [skew_idx + l])
                        for l in range(16)]
                while len(cols) > 1:
                    cols = [a + b for a, b in zip(cols[::2], cols[1::2])]
                att_v[pl.ds(nb * 16, 16)] = cols[0]
                return carry

            lax.fori_loop(0, nng, dot_body, 0)

            accs = [att_v[pl.ds(ng * 16, 16)] for ng in range(nng)]
            m8 = accs[0]
            for ng in range(1, nng):
                m8 = jnp.maximum(m8, accs[ng])
            m = jnp.max(m8)
            exps = [jnp.exp(a - m) for a in accs]
            s8 = exps[0]
            for ng in range(1, nng):
                s8 = s8 + exps[ng]
            s = jnp.sum(s8)
            for ng in range(nng):
                att_v[pl.ds(ng * 16, 16)] = exps[ng] / s

            def agg_body(nb, aggs):
                ablk = att_v[pl.ds(nb * 16, 16)]
                n0 = nb * 16
                for j in range(16):
                    an = ablk[j]
                    aggs = tuple(
                        aggs[dg] + an * ebuf[0, n0 + j, pl.ds(dg * 16, 16)]
                        for dg in range(ndg))
                return aggs

            aggs = lax.fori_loop(
                0, nng, agg_body, tuple([zero16f] * ndg))
            for dg in range(ndg):
                obuf[pl.ds(t * d + dg * 16, 16)] = aggs[dg]

        start(0, 0)
        start(1, 1)

        def pair_body(g, carry):
            t0 = 2 * g
            wait(0)
            compute_token(t0, ebuf_a)
            start(t0 + 2, 0)
            wait(1)
            compute_token(t0 + 1, ebuf_b)

            @pl.when(t0 + 3 < tpw)
            def _():
                start(t0 + 3, 1)

            return carry

        lax.fori_loop(0, (tpw - 1) // 2, pair_body, 0)
        wait(0)
        compute_token(tpw - 1, ebuf_a)

        pltpu.sync_copy(obuf, out_hbm.at[pl.ds(base * d, tpw * d)])

    return body(ts, table, v_flat)


def kernel(internal_emb, timestamps, time_list, ext_embeddings,
           time_to_embeddings, W_att, b_att):
    bs, seq, d = internal_emb.shape
    t_max, nextn, _ = ext_embeddings.shape
    internal_flat = internal_emb.reshape(bs * seq, d)
    ts_flat = timestamps.reshape(-1).astype(jnp.int32)
    v = _v_matmul(internal_flat, W_att)
    out = _sc_aggregate(ts_flat, ext_embeddings, v.reshape(-1), t_max, d)
    return out.reshape(bs, seq, d)
